# baseline (speedup 1.0000x reference)
import functools

import jax
import jax.numpy as jnp
from jax import lax
from jax.experimental import pallas as pl
from jax.experimental.pallas import tpu as pltpu

_C = 256
_A = 9
_NCLS = 80

_LEVEL_CFG = {
    80: (80, 8, 16, 2),
    40: (40, 8, 8, 4),
    20: (24, 10, 20, 5),
    10: (16, 10, 10, 5),
    5: (8, 5, 5, 5),
}


def _head_kernel(x_ref, tw_ref, tb_ref, ow_ref, ob_ref, out_ref, xb, pb, *,
                 S, W, Wp, MB, RB, MBo, Do):
    j = pl.program_id(1)

    @pl.when(j == 0)
    def _tower():
        for buf in (xb, pb):
            buf[0:1, :, :] = jnp.zeros((1, Wp + 2, _C), jnp.bfloat16)
            buf[S + 1:S + 2, :, :] = jnp.zeros((1, Wp + 2, _C), jnp.bfloat16)
            buf[:, 0:1, :] = jnp.zeros((S + 2, 1, _C), jnp.bfloat16)
            buf[:, W + 1:Wp + 2, :] = jnp.zeros((S + 2, Wp + 1 - W, _C),
                                                jnp.bfloat16)
        xb[1:S + 1, 1:W + 1, :] = x_ref[0]
        for layer in range(4):
            src, dst = (xb, pb) if layer % 2 == 0 else (pb, xb)
            wks = [[tw_ref[layer, ky, kx] for kx in range(3)]
                   for ky in range(3)]
            bias = tb_ref[layer]

            def chunk(ci, carry, src=src, dst=dst, wks=wks, bias=bias):
                r0 = ci * MB
                acc = jnp.zeros((MB * Wp, _C), jnp.float32)
                for ky in range(3):
                    for kx in range(3):
                        lhs = src[pl.ds(r0 + ky, MB),
                                  pl.ds(kx, Wp), :].reshape(MB * Wp, _C)
                        acc = acc + jnp.dot(
                            lhs, wks[ky][kx],
                            preferred_element_type=jnp.float32)
                y = jnp.maximum(acc + bias, 0.0).astype(jnp.bfloat16)
                y = y.reshape(MB, Wp, _C)
                if Wp != W:
                    col = lax.broadcasted_iota(jnp.int32, (MB, Wp, _C), 1)
                    y = jnp.where(col < W, y, jnp.bfloat16(0))
                dst[pl.ds(r0 + 1, MB), pl.ds(1, Wp), :] = y
                return carry

            lax.fori_loop(0, S // MB, chunk, 0)

    ows = [[ow_ref[ky, kx] for kx in range(3)] for ky in range(3)]
    ob = ob_ref[...]

    def ochunk(ci, carry):
        r0 = j * RB + ci * MBo
        acc = jnp.zeros((MBo * Wp, Do), jnp.float32)
        for ky in range(3):
            for kx in range(3):
                lhs = xb[pl.ds(r0 + ky, MBo),
                         pl.ds(kx, Wp), :].reshape(MBo * Wp, _C)
                acc = acc + jnp.dot(lhs, ows[ky][kx],
                                    preferred_element_type=jnp.float32)
        out_ref[0, pl.ds(ci * MBo, MBo), :, :] = (acc + ob).reshape(
            MBo, Wp, Do)
        return carry

    lax.fori_loop(0, RB // MBo, ochunk, 0)


def _run_head(x, tw, tb, ow, obias, *, S, W, Wp, MB, RB, MBo, Do, name,
              interpret=False):
    N = x.shape[0]
    NB = S // RB
    kern = functools.partial(_head_kernel, S=S, W=W, Wp=Wp, MB=MB, RB=RB,
                             MBo=MBo, Do=Do)
    return pl.pallas_call(
        kern,
        grid=(N, NB),
        in_specs=[
            pl.BlockSpec((1, S, W, _C), lambda n, j: (n, 0, 0, 0)),
            pl.BlockSpec((4, 3, 3, _C, _C), lambda n, j: (0, 0, 0, 0, 0)),
            pl.BlockSpec((4, 1, _C), lambda n, j: (0, 0, 0)),
            pl.BlockSpec((3, 3, _C, Do), lambda n, j: (0, 0, 0, 0)),
            pl.BlockSpec((1, Do), lambda n, j: (0, 0)),
        ],
        out_specs=pl.BlockSpec((1, RB, Wp, Do), lambda n, j: (n, j, 0, 0)),
        out_shape=jax.ShapeDtypeStruct((N, S, Wp, Do), jnp.float32),
        scratch_shapes=[
            pltpu.VMEM((S + 2, Wp + 2, _C), jnp.bfloat16),
            pltpu.VMEM((S + 2, Wp + 2, _C), jnp.bfloat16),
        ],
        compiler_params=pltpu.CompilerParams(
            dimension_semantics=("parallel", "arbitrary"),
            vmem_limit_bytes=100 * 1024 * 1024,
        ),
        name=name,
        interpret=interpret,
    )(x, tw, tb, ow, obias)


def kernel(x0, x1, x2, x3, x4,
           cls_conv_w, cls_conv_b, cls_out_w, cls_out_b,
           reg_conv_w, reg_conv_b, reg_out_w, reg_out_b):
    feats = [x0, x1, x2, x3, x4]
    N = x0.shape[0]

    def prep_head(conv_w, conv_b, out_w, out_b):
        tw = jnp.transpose(conv_w, (0, 3, 4, 2, 1)).astype(jnp.bfloat16)
        tb = conv_b.astype(jnp.float32).reshape(4, 1, _C)
        ow = jnp.transpose(out_w, (2, 3, 1, 0)).astype(jnp.bfloat16)
        obias = out_b.astype(jnp.float32).reshape(1, -1)
        return tw, tb, ow, obias

    cls_p = prep_head(cls_conv_w, cls_conv_b, cls_out_w, cls_out_b)
    reg_p = prep_head(reg_conv_w, reg_conv_b, reg_out_w, reg_out_b)

    cls_parts, reg_parts = [], []
    for f in feats:
        S = f.shape[2]
        Wp, MB, RB, MBo = _LEVEL_CFG[S]
        xh = jnp.transpose(f, (0, 2, 3, 1)).astype(jnp.bfloat16)
        oc = _run_head(xh, *cls_p, S=S, W=S, Wp=Wp, MB=MB, RB=RB, MBo=MBo,
                       Do=_A * _NCLS, name=f"retina_cls_{S}")
        og = _run_head(xh, *reg_p, S=S, W=S, Wp=Wp, MB=MB, RB=RB, MBo=MBo,
                       Do=_A * 4, name=f"retina_reg_{S}")
        if Wp != S:
            oc = oc[:, :, :S, :]
            og = og[:, :, :S, :]
        cls_parts.append(oc.reshape(N, S * S * _A, _NCLS))
        reg_parts.append(og.reshape(N, S * S * _A, 4))
    return (jnp.concatenate(cls_parts, axis=1),
            jnp.concatenate(reg_parts, axis=1))
